# baseline (speedup 1.0000x reference)
import functools

import jax
import jax.numpy as jnp
from jax import lax
from jax.experimental import pallas as pl
from jax.experimental.pallas import tpu as pltpu
from jax.experimental.pallas import tpu_sc as plsc

N = 4096
D = 4096
M = 8
B = 128
NBMAX = 48
NBG = N // B + 7
PN = NBG * B
NO = 4
DO = D // NO

NW = 32
TPW = N // NW
GPW = TPW // 16


def _dyn_gather(table, idx):
    dnums = lax.GatherDimensionNumbers(
        offset_dims=(), collapsed_slice_dims=(0,), start_index_map=(0,))
    return lax.gather(table, idx[:, None], dnums, (1,),
                      mode=lax.GatherScatterMode.PROMISE_IN_BOUNDS)


def _route_scatter_body(in_hbm, ids_hbm, xs_hbm, slot_hbm, bm_hbm, xi_hbm,
                        ids_v, slot_v, buf, sem):
    wid = lax.axis_index("s") * 2 + lax.axis_index("c")
    lane = lax.iota(jnp.int32, 16)

    pltpu.sync_copy(ids_hbm, ids_v)

    def hist_step(g, carry):
        counts, pfx = carry
        pfx = jnp.where(g == wid * GPW, counts, pfx)
        v = ids_v[pl.ds(g * 16, 16)]
        for m in range(M):
            pc = jnp.sum((v == m).astype(jnp.int32))
            counts = counts + jnp.where(lane == m, pc, 0)
        return counts, pfx

    zero = jnp.zeros((16,), jnp.int32)
    counts, pfx = lax.fori_loop(0, N // 16, hist_step, (zero, zero))

    inlane = lane < M
    nb = jnp.where(inlane, (counts + (B - 1)) // B, 0)
    csum = jnp.cumsum(nb)
    sb = csum - nb
    rowbase = sb * B
    nb_used_s = _dyn_gather(csum, jnp.full((16,), M - 1, jnp.int32))

    cnt = pfx
    for k in range(GPW):
        g = wid * GPW + k
        v = ids_v[pl.ds(g * 16, 16)]
        psame = zero
        for m in range(M):
            mask = v == m
            mi = mask.astype(jnp.int32)
            ex = jnp.cumsum(mi) - mi
            psame = jnp.where(mask, ex, psame)
        base_v = _dyn_gather(rowbase, v)
        run_v = _dyn_gather(cnt, v)
        slot_v[k, :] = base_v + run_v + psame
        for m in range(M):
            pc = jnp.sum((v == m).astype(jnp.int32))
            cnt = cnt + jnp.where(lane == m, pc, 0)

    pltpu.sync_copy(slot_v, slot_hbm.at[pl.ds(wid * GPW, GPW)])

    for k in range(GPW):
        pltpu.sync_copy(in_hbm.at[pl.ds(wid * TPW + k * 16, 16)], buf)
        pltpu.async_copy(buf, xs_hbm.at[slot_v.at[k]], sem).wait()

    @pl.when(wid == 0)
    def _():
        lastm = jnp.max(jnp.where(inlane & (nb > 0), lane, 0))
        for r in range(NBMAX // 16):
            iv = lane + r * 16
            bmv = zero
            for m in range(1, M):
                sbm = _dyn_gather(sb, jnp.full((16,), m, jnp.int32))
                bmv = bmv + (iv >= sbm).astype(jnp.int32)
            bmv = jnp.where(iv < nb_used_s, bmv, lastm)
            slot_v[r, :] = bmv
            slot_v[r + NBMAX // 16, :] = jnp.minimum(iv, nb_used_s - 1)
        pltpu.sync_copy(slot_v.at[pl.ds(0, NBMAX // 16)],
                        bm_hbm.at[pl.ds(0, NBMAX // 16)])
        pltpu.sync_copy(slot_v.at[pl.ds(NBMAX // 16, NBMAX // 16)],
                        xi_hbm.at[pl.ds(0, NBMAX // 16)])


_route_scatter = functools.partial(
    pl.kernel,
    out_type=(
        jax.ShapeDtypeStruct((PN, D), jnp.float32),
        jax.ShapeDtypeStruct((N // 16, 16), jnp.int32),
        jax.ShapeDtypeStruct((NBMAX // 16, 16), jnp.int32),
        jax.ShapeDtypeStruct((NBMAX // 16, 16), jnp.int32),
    ),
    mesh=plsc.VectorSubcoreMesh(core_axis_name="c", subcore_axis_name="s"),
    compiler_params=pltpu.CompilerParams(needs_layout_passes=False),
    scratch_types=[
        pltpu.VMEM((N,), jnp.int32),
        pltpu.VMEM((GPW, 16), jnp.int32),
        pltpu.VMEM((16, D), jnp.float32),
        pltpu.SemaphoreType.DMA,
    ],
)(_route_scatter_body)


def _gather_back_body(y_hbm, slot_hbm, out_hbm, slot_v, buf, sem):
    wid = lax.axis_index("s") * 2 + lax.axis_index("c")
    pltpu.sync_copy(slot_hbm.at[pl.ds(wid * GPW, GPW)], slot_v)
    for k in range(GPW):
        pltpu.async_copy(y_hbm.at[slot_v.at[k]], buf, sem).wait()
        pltpu.sync_copy(buf, out_hbm.at[pl.ds(wid * TPW + k * 16, 16)])


_gather_back = functools.partial(
    pl.kernel,
    out_type=jax.ShapeDtypeStruct((N, D), jnp.float32),
    mesh=plsc.VectorSubcoreMesh(core_axis_name="c", subcore_axis_name="s"),
    compiler_params=pltpu.CompilerParams(needs_layout_passes=False),
    scratch_types=[
        pltpu.VMEM((GPW, 16), jnp.int32),
        pltpu.VMEM((16, D), jnp.float32),
        pltpu.SemaphoreType.DMA,
    ],
)(_gather_back_body)


def _mm_body(bm_ref, xi_ref, x_ref, w_ref, b_ref, o_ref, wbf):
    i = pl.program_id(1)
    prev = bm_ref[jnp.maximum(i - 1, 0)]
    changed = (i == 0) | (bm_ref[i] != prev)

    @pl.when(changed)
    def _():
        wbf[...] = w_ref[0].astype(jnp.bfloat16)

    @pl.when(xi_ref[i] == i)
    def _():
        x = x_ref[...].astype(jnp.bfloat16)
        acc = lax.dot_general(x, wbf[...], (((1,), (1,)), ((), ())),
                              preferred_element_type=jnp.float32)
        o_ref[...] = acc + b_ref[0, 0][None, :]


def _grouped_matmul(bm, xi, xs, W, b3):
    grid = (NO, NBG)
    spec = pltpu.PrefetchScalarGridSpec(
        num_scalar_prefetch=2,
        grid=grid,
        in_specs=[
            pl.BlockSpec((B, D), lambda j, i, bm, xi: (xi[i], 0)),
            pl.BlockSpec((1, DO, D), lambda j, i, bm, xi: (bm[i], j, 0)),
            pl.BlockSpec((1, 1, DO), lambda j, i, bm, xi: (bm[i], 0, j)),
        ],
        out_specs=pl.BlockSpec((B, DO), lambda j, i, bm, xi: (i, j)),
        scratch_shapes=[pltpu.VMEM((DO, D), jnp.bfloat16)],
    )
    return pl.pallas_call(
        _mm_body,
        grid_spec=spec,
        out_shape=jax.ShapeDtypeStruct((PN, D), jnp.float32),
        compiler_params=pltpu.CompilerParams(
            dimension_semantics=("arbitrary", "arbitrary")),
    )(bm, xi, xs, W, b3)


def kernel(in_feats, module_ids, W, b):
    xs, slot, bm, xi = _route_scatter(in_feats, module_ids)
    y = _grouped_matmul(bm.reshape(-1), xi.reshape(-1), xs, W,
                        b.reshape(M, 1, D))
    return _gather_back(y, slot)


# speedup vs baseline: 1.4699x; 1.4699x over previous
import functools

import jax
import jax.numpy as jnp
from jax import lax
from jax.experimental import pallas as pl
from jax.experimental.pallas import tpu as pltpu
from jax.experimental.pallas import tpu_sc as plsc

N = 4096
D = 4096
M = 8
B = 256
NBMAX = 32
NBG = N // B + 7
PN = NBG * B
NO = 4
DO = D // NO

NW = 32
TPW = N // NW
GPW = TPW // 16


def _dyn_gather(table, idx):
    dnums = lax.GatherDimensionNumbers(
        offset_dims=(), collapsed_slice_dims=(0,), start_index_map=(0,))
    return lax.gather(table, idx[:, None], dnums, (1,),
                      mode=lax.GatherScatterMode.PROMISE_IN_BOUNDS)


def _route_scatter_body(in_hbm, ids_hbm, xs_hbm, slot_hbm, bm_hbm, xi_hbm,
                        ids_v, slot_v, buf0, buf1, si0, si1, so0, so1):
    wid = lax.axis_index("s") * 2 + lax.axis_index("c")
    lane = lax.iota(jnp.int32, 16)
    bufs, sin, sout = (buf0, buf1), (si0, si1), (so0, so1)
    H = D // 2

    def start_in(t):
        k, c = divmod(t, 2)
        return pltpu.async_copy(
            in_hbm.at[pl.ds(wid * TPW + k * 16, 16), pl.ds(c * H, H)],
            bufs[t % 2], sin[t % 2])

    in_h = [None] * (2 * GPW)
    in_h[0] = start_in(0)
    in_h[1] = start_in(1)

    pltpu.sync_copy(ids_hbm, ids_v)

    def hist_step(g, carry):
        counts, pfx = carry
        pfx = jnp.where(g == wid * GPW, counts, pfx)
        v = ids_v[pl.ds(g * 16, 16)]
        for m in range(M):
            pc = jnp.sum((v == m).astype(jnp.int32))
            counts = counts + jnp.where(lane == m, pc, 0)
        return counts, pfx

    zero = jnp.zeros((16,), jnp.int32)
    counts, pfx = lax.fori_loop(0, N // 16, hist_step, (zero, zero))

    inlane = lane < M
    nb = jnp.where(inlane, (counts + (B - 1)) // B, 0)
    csum = jnp.cumsum(nb)
    sb = csum - nb
    rowbase = sb * B
    nb_used_s = _dyn_gather(csum, jnp.full((16,), M - 1, jnp.int32))

    cnt = pfx
    for k in range(GPW):
        g = wid * GPW + k
        v = ids_v[pl.ds(g * 16, 16)]
        psame = zero
        for m in range(M):
            mask = v == m
            mi = mask.astype(jnp.int32)
            ex = jnp.cumsum(mi) - mi
            psame = jnp.where(mask, ex, psame)
        base_v = _dyn_gather(rowbase, v)
        run_v = _dyn_gather(cnt, v)
        slot_v[k, :] = base_v + run_v + psame
        for m in range(M):
            pc = jnp.sum((v == m).astype(jnp.int32))
            cnt = cnt + jnp.where(lane == m, pc, 0)

    pltpu.sync_copy(slot_v, slot_hbm.at[pl.ds(wid * GPW, GPW)])

    NT = 2 * GPW
    out_h = [None] * NT
    for t in range(NT):
        k, c = divmod(t, 2)
        in_h[t].wait()
        out_h[t] = pltpu.async_copy(
            bufs[t % 2],
            xs_hbm.at[slot_v.at[k], pl.ds(c * H, H)], sout[t % 2])
        if t + 2 < NT:
            out_h[t].wait()
            in_h[t + 2] = start_in(t + 2)
    out_h[NT - 2].wait()
    out_h[NT - 1].wait()

    @pl.when(wid == 0)
    def _():
        lastm = jnp.max(jnp.where(inlane & (nb > 0), lane, 0))
        for r in range(NBMAX // 16):
            iv = lane + r * 16
            bmv = zero
            for m in range(1, M):
                sbm = _dyn_gather(sb, jnp.full((16,), m, jnp.int32))
                bmv = bmv + (iv >= sbm).astype(jnp.int32)
            bmv = jnp.where(iv < nb_used_s, bmv, lastm)
            slot_v[r, :] = bmv
            slot_v[r + NBMAX // 16, :] = jnp.minimum(iv, nb_used_s - 1)
        pltpu.sync_copy(slot_v.at[pl.ds(0, NBMAX // 16)],
                        bm_hbm.at[pl.ds(0, NBMAX // 16)])
        pltpu.sync_copy(slot_v.at[pl.ds(NBMAX // 16, NBMAX // 16)],
                        xi_hbm.at[pl.ds(0, NBMAX // 16)])


_route_scatter = functools.partial(
    pl.kernel,
    out_type=(
        jax.ShapeDtypeStruct((PN, D), jnp.float32),
        jax.ShapeDtypeStruct((N // 16, 16), jnp.int32),
        jax.ShapeDtypeStruct((NBMAX // 16, 16), jnp.int32),
        jax.ShapeDtypeStruct((NBMAX // 16, 16), jnp.int32),
    ),
    mesh=plsc.VectorSubcoreMesh(core_axis_name="c", subcore_axis_name="s"),
    compiler_params=pltpu.CompilerParams(needs_layout_passes=False),
    scratch_types=[
        pltpu.VMEM((N,), jnp.int32),
        pltpu.VMEM((GPW, 16), jnp.int32),
        pltpu.VMEM((16, D // 2), jnp.float32),
        pltpu.VMEM((16, D // 2), jnp.float32),
        pltpu.SemaphoreType.DMA,
        pltpu.SemaphoreType.DMA,
        pltpu.SemaphoreType.DMA,
        pltpu.SemaphoreType.DMA,
    ],
)(_route_scatter_body)


def _gather_back_body(y_hbm, slot_hbm, out_hbm, slot_v,
                      buf0, buf1, si0, si1, so0, so1):
    wid = lax.axis_index("s") * 2 + lax.axis_index("c")
    bufs, sin, sout = (buf0, buf1), (si0, si1), (so0, so1)
    H = D // 2
    pltpu.sync_copy(slot_hbm.at[pl.ds(wid * GPW, GPW)], slot_v)

    def start_gather(t):
        k, c = divmod(t, 2)
        return pltpu.async_copy(
            y_hbm.at[slot_v.at[k], pl.ds(c * H, H)], bufs[t % 2], sin[t % 2])

    NT = 2 * GPW
    in_h = [None] * NT
    out_h = [None] * NT
    in_h[0] = start_gather(0)
    in_h[1] = start_gather(1)
    for t in range(NT):
        k, c = divmod(t, 2)
        in_h[t].wait()
        out_h[t] = pltpu.async_copy(
            bufs[t % 2],
            out_hbm.at[pl.ds(wid * TPW + k * 16, 16), pl.ds(c * H, H)],
            sout[t % 2])
        if t + 2 < NT:
            out_h[t].wait()
            in_h[t + 2] = start_gather(t + 2)
    out_h[NT - 2].wait()
    out_h[NT - 1].wait()


_gather_back = functools.partial(
    pl.kernel,
    out_type=jax.ShapeDtypeStruct((N, D), jnp.float32),
    mesh=plsc.VectorSubcoreMesh(core_axis_name="c", subcore_axis_name="s"),
    compiler_params=pltpu.CompilerParams(needs_layout_passes=False),
    scratch_types=[
        pltpu.VMEM((GPW, 16), jnp.int32),
        pltpu.VMEM((16, D // 2), jnp.float32),
        pltpu.VMEM((16, D // 2), jnp.float32),
        pltpu.SemaphoreType.DMA,
        pltpu.SemaphoreType.DMA,
        pltpu.SemaphoreType.DMA,
        pltpu.SemaphoreType.DMA,
    ],
)(_gather_back_body)


def _mm_body(bm_ref, xi_ref, x_ref, w_ref, b_ref, o_ref):
    i = pl.program_id(1)

    @pl.when(xi_ref[i] == i)
    def _():
        x = x_ref[...].astype(jnp.bfloat16)
        acc = lax.dot_general(x, w_ref[0], (((1,), (1,)), ((), ())),
                              preferred_element_type=jnp.float32)
        o_ref[...] = acc + b_ref[0, 0][None, :]


def _grouped_matmul(bm, xi, xs, W, b3):
    grid = (NO, NBG)
    spec = pltpu.PrefetchScalarGridSpec(
        num_scalar_prefetch=2,
        grid=grid,
        in_specs=[
            pl.BlockSpec((B, D), lambda j, i, bm, xi: (xi[i], 0)),
            pl.BlockSpec((1, DO, D), lambda j, i, bm, xi: (bm[i], j, 0)),
            pl.BlockSpec((1, 1, DO), lambda j, i, bm, xi: (bm[i], 0, j)),
        ],
        out_specs=pl.BlockSpec((B, DO), lambda j, i, bm, xi: (i, j)),
    )
    return pl.pallas_call(
        _mm_body,
        grid_spec=spec,
        out_shape=jax.ShapeDtypeStruct((PN, D), jnp.float32),
        compiler_params=pltpu.CompilerParams(
            dimension_semantics=("arbitrary", "arbitrary")),
    )(bm, xi, xs, W, b3)


def kernel(in_feats, module_ids, W, b):
    xs, slot, bm, xi = _route_scatter(in_feats, module_ids)
    y = _grouped_matmul(bm.reshape(-1), xi.reshape(-1), xs, W,
                        b.reshape(M, 1, D))
    return _gather_back(y, slot)


# speedup vs baseline: 1.6195x; 1.1018x over previous
import functools

import jax
import jax.numpy as jnp
from jax import lax
from jax.experimental import pallas as pl
from jax.experimental.pallas import tpu as pltpu
from jax.experimental.pallas import tpu_sc as plsc

N = 4096
D = 4096
M = 8
B = 256
NBMAX = 32
NBG = N // B + 7
PN = NBG * B
NO = 4
DO = D // NO

NW = 32
TPW = N // NW
GPW = TPW // 16


def _dyn_gather(table, idx):
    dnums = lax.GatherDimensionNumbers(
        offset_dims=(), collapsed_slice_dims=(0,), start_index_map=(0,))
    return lax.gather(table, idx[:, None], dnums, (1,),
                      mode=lax.GatherScatterMode.PROMISE_IN_BOUNDS)


def _route_scatter_body(in_hbm, ids_hbm, xs_hbm, slot_hbm, bm_hbm, xi_hbm,
                        nxt_hbm, ids_v, slot_v, buf0, buf1, si0, si1,
                        so0, so1):
    wid = lax.axis_index("s") * 2 + lax.axis_index("c")
    lane = lax.iota(jnp.int32, 16)
    bufs, sin, sout = (buf0, buf1), (si0, si1), (so0, so1)
    H = D // 2

    def start_in(t):
        k, c = divmod(t, 2)
        return pltpu.async_copy(
            in_hbm.at[pl.ds(wid * TPW + k * 16, 16), pl.ds(c * H, H)],
            bufs[t % 2], sin[t % 2])

    in_h = [None] * (2 * GPW)
    in_h[0] = start_in(0)
    in_h[1] = start_in(1)

    pltpu.sync_copy(ids_hbm, ids_v)

    def hist_step(g, carry):
        counts, pfx = carry
        pfx = jnp.where(g == wid * GPW, counts, pfx)
        v = ids_v[pl.ds(g * 16, 16)]
        for m in range(M):
            pc = jnp.sum((v == m).astype(jnp.int32))
            counts = counts + jnp.where(lane == m, pc, 0)
        return counts, pfx

    zero = jnp.zeros((16,), jnp.int32)
    counts, pfx = lax.fori_loop(0, N // 16, hist_step, (zero, zero))

    inlane = lane < M
    nb = jnp.where(inlane, (counts + (B - 1)) // B, 0)
    csum = jnp.cumsum(nb)
    sb = csum - nb
    rowbase = sb * B
    nb_used_s = _dyn_gather(csum, jnp.full((16,), M - 1, jnp.int32))

    cnt = pfx
    for k in range(GPW):
        g = wid * GPW + k
        v = ids_v[pl.ds(g * 16, 16)]
        psame = zero
        for m in range(M):
            mask = v == m
            mi = mask.astype(jnp.int32)
            ex = jnp.cumsum(mi) - mi
            psame = jnp.where(mask, ex, psame)
        base_v = _dyn_gather(rowbase, v)
        run_v = _dyn_gather(cnt, v)
        slot_v[k, :] = base_v + run_v + psame
        for m in range(M):
            pc = jnp.sum((v == m).astype(jnp.int32))
            cnt = cnt + jnp.where(lane == m, pc, 0)

    pltpu.sync_copy(slot_v, slot_hbm.at[pl.ds(wid * GPW, GPW)])

    NT = 2 * GPW
    out_h = [None] * NT
    for t in range(NT):
        k, c = divmod(t, 2)
        in_h[t].wait()
        out_h[t] = pltpu.async_copy(
            bufs[t % 2],
            xs_hbm.at[slot_v.at[k], pl.ds(c * H, H)], sout[t % 2])
        if t + 2 < NT:
            out_h[t].wait()
            in_h[t + 2] = start_in(t + 2)
    out_h[NT - 2].wait()
    out_h[NT - 1].wait()

    @pl.when(wid == 0)
    def _():
        lastm = jnp.max(jnp.where(inlane & (nb > 0), lane, 0))
        fm = jnp.min(jnp.where(inlane & (nb > 0), lane, 99))
        npres = zero + fm
        for mm in range(7, 0, -1):
            nbm = _dyn_gather(nb, jnp.full((16,), mm, jnp.int32))
            npres = jnp.where((lane < mm) & (nbm > 0), mm, npres)
        NR = NBMAX // 16
        for r in range(NR):
            iv = lane + r * 16
            bmv = zero
            for m in range(1, M):
                sbm = _dyn_gather(sb, jnp.full((16,), m, jnp.int32))
                bmv = bmv + (iv >= sbm).astype(jnp.int32)
            bmv = jnp.where(iv < nb_used_s, bmv, lastm)
            slot_v[r, :] = bmv
            slot_v[r + NR, :] = jnp.minimum(iv, nb_used_s - 1)
            nxtv = _dyn_gather(npres, bmv) + 8 * (bmv == lastm).astype(jnp.int32)
            slot_v[r + 2 * NR, :] = nxtv
        pltpu.sync_copy(slot_v.at[pl.ds(0, NR)], bm_hbm.at[pl.ds(0, NR)])
        pltpu.sync_copy(slot_v.at[pl.ds(NR, NR)], xi_hbm.at[pl.ds(0, NR)])
        pltpu.sync_copy(slot_v.at[pl.ds(2 * NR, NR)], nxt_hbm.at[pl.ds(0, NR)])


_route_scatter = functools.partial(
    pl.kernel,
    out_type=(
        jax.ShapeDtypeStruct((PN, D), jnp.float32),
        jax.ShapeDtypeStruct((N // 16, 16), jnp.int32),
        jax.ShapeDtypeStruct((NBMAX // 16, 16), jnp.int32),
        jax.ShapeDtypeStruct((NBMAX // 16, 16), jnp.int32),
        jax.ShapeDtypeStruct((NBMAX // 16, 16), jnp.int32),
    ),
    mesh=plsc.VectorSubcoreMesh(core_axis_name="c", subcore_axis_name="s"),
    compiler_params=pltpu.CompilerParams(needs_layout_passes=False),
    scratch_types=[
        pltpu.VMEM((N,), jnp.int32),
        pltpu.VMEM((GPW, 16), jnp.int32),
        pltpu.VMEM((16, D // 2), jnp.float32),
        pltpu.VMEM((16, D // 2), jnp.float32),
        pltpu.SemaphoreType.DMA,
        pltpu.SemaphoreType.DMA,
        pltpu.SemaphoreType.DMA,
        pltpu.SemaphoreType.DMA,
    ],
)(_route_scatter_body)


def _gather_back_body(y_hbm, slot_hbm, out_hbm, slot_v,
                      buf0, buf1, si0, si1, so0, so1):
    wid = lax.axis_index("s") * 2 + lax.axis_index("c")
    bufs, sin, sout = (buf0, buf1), (si0, si1), (so0, so1)
    H = D // 2
    pltpu.sync_copy(slot_hbm.at[pl.ds(wid * GPW, GPW)], slot_v)

    def start_gather(t):
        k, c = divmod(t, 2)
        return pltpu.async_copy(
            y_hbm.at[slot_v.at[k], pl.ds(c * H, H)], bufs[t % 2], sin[t % 2])

    NT = 2 * GPW
    in_h = [None] * NT
    out_h = [None] * NT
    in_h[0] = start_gather(0)
    in_h[1] = start_gather(1)
    for t in range(NT):
        k, c = divmod(t, 2)
        in_h[t].wait()
        out_h[t] = pltpu.async_copy(
            bufs[t % 2],
            out_hbm.at[pl.ds(wid * TPW + k * 16, 16), pl.ds(c * H, H)],
            sout[t % 2])
        if t + 2 < NT:
            out_h[t].wait()
            in_h[t + 2] = start_gather(t + 2)
    out_h[NT - 2].wait()
    out_h[NT - 1].wait()


_gather_back = functools.partial(
    pl.kernel,
    out_type=jax.ShapeDtypeStruct((N, D), jnp.float32),
    mesh=plsc.VectorSubcoreMesh(core_axis_name="c", subcore_axis_name="s"),
    compiler_params=pltpu.CompilerParams(needs_layout_passes=False),
    scratch_types=[
        pltpu.VMEM((GPW, 16), jnp.int32),
        pltpu.VMEM((16, D // 2), jnp.float32),
        pltpu.VMEM((16, D // 2), jnp.float32),
        pltpu.SemaphoreType.DMA,
        pltpu.SemaphoreType.DMA,
        pltpu.SemaphoreType.DMA,
        pltpu.SemaphoreType.DMA,
    ],
)(_gather_back_body)


def _mm_body(bm_ref, xi_ref, nxt_ref, x_ref, w_hbm, b_ref, o_ref,
             wbuf, sm, sems):
    j = pl.program_id(0)
    i = pl.program_id(1)
    newrun = (i == 0) | (bm_ref[i] != bm_ref[jnp.maximum(i - 1, 0)])

    @pl.when((j == 0) & (i == 0))
    def _():
        sm[0] = 0
        pltpu.make_async_copy(
            w_hbm.at[bm_ref[0], pl.ds(0, DO)], wbuf.at[0], sems.at[0]).start()

    @pl.when(newrun)
    def _():
        cur = sm[0]
        slot = lax.rem(cur, 2)
        pltpu.make_async_copy(
            w_hbm.at[0, pl.ds(0, DO)], wbuf.at[slot], sems.at[slot]).wait()
        nx = nxt_ref[i]
        nm = lax.rem(nx, 8)
        jt = j + nx // 8

        @pl.when(jt < NO)
        def _():
            pltpu.make_async_copy(
                w_hbm.at[nm, pl.ds(jt * DO, DO)], wbuf.at[1 - slot],
                sems.at[1 - slot]).start()

        sm[0] = cur + 1
        sm[1] = slot

    @pl.when(xi_ref[i] == i)
    def _():
        x = x_ref[...].astype(jnp.bfloat16)
        acc = lax.dot_general(x, wbuf[sm[1]], (((1,), (1,)), ((), ())),
                              preferred_element_type=jnp.float32)
        o_ref[...] = acc + b_ref[0, 0][None, :]


def _grouped_matmul(bm, xi, nxt, xs, W, b3):
    grid = (NO, NBG)
    spec = pltpu.PrefetchScalarGridSpec(
        num_scalar_prefetch=3,
        grid=grid,
        in_specs=[
            pl.BlockSpec((B, D), lambda j, i, bm, xi, nxt: (xi[i], 0)),
            pl.BlockSpec(memory_space=pltpu.MemorySpace.HBM),
            pl.BlockSpec((1, 1, DO), lambda j, i, bm, xi, nxt: (bm[i], 0, j)),
        ],
        out_specs=pl.BlockSpec((B, DO), lambda j, i, bm, xi, nxt: (i, j)),
        scratch_shapes=[
            pltpu.VMEM((2, DO, D), jnp.float32),
            pltpu.SMEM((2,), jnp.int32),
            pltpu.SemaphoreType.DMA((2,)),
        ],
    )
    return pl.pallas_call(
        _mm_body,
        grid_spec=spec,
        out_shape=jax.ShapeDtypeStruct((PN, D), jnp.float32),
        compiler_params=pltpu.CompilerParams(
            dimension_semantics=("arbitrary", "arbitrary")),
    )(bm, xi, nxt, xs, W, b3)


def kernel(in_feats, module_ids, W, b):
    xs, slot, bm, xi, nxt = _route_scatter(in_feats, module_ids)
    y = _grouped_matmul(bm.reshape(-1), xi.reshape(-1), nxt.reshape(-1),
                        xs, W, b.reshape(M, 1, D))
    return _gather_back(y, slot)
